# Initial kernel scaffold; baseline (speedup 1.0000x reference)
import jax
import jax.numpy as jnp
from jax.experimental import pallas as pl


def kernel(signal, bary_verts, bary_weights, kernel):
    raise NotImplementedError("write your pallas kernel here")


# speedup vs baseline: 2.2729x; 2.2729x over previous
import functools

import jax
import jax.numpy as jnp
import numpy as np
from jax import lax
from jax.experimental import pallas as pl
from jax.experimental.pallas import tpu as pltpu
from jax.experimental.pallas import tpu_sc as plsc

N = 10000
D = 128
D_OUT = 32
KR, KT = 2, 8
K = KR * KT
NK = N * K
NW = 32
C = 64
NCHUNKS = NK // C
TPW = (NCHUNKS + NW - 1) // NW


def _sc_pullback(signal, idx3, w3):
    mesh = plsc.VectorSubcoreMesh(core_axis_name="c", subcore_axis_name="s")

    @functools.partial(
        pl.kernel,
        out_type=jax.ShapeDtypeStruct((NK, D), jnp.float32),
        mesh=mesh,
        scratch_types=[
            pltpu.VMEM((3, C), jnp.int32),
            pltpu.VMEM((3, C), jnp.float32),
            pltpu.VMEM((3, C, D), jnp.float32),
            pltpu.VMEM((C, D), jnp.float32),
            pltpu.SemaphoreType.DMA,
        ],
    )
    def body(signal_hbm, idx_hbm, w_hbm, out_hbm, idx_v, w_v, rows_v, acc_v, sem):
        wid = lax.axis_index("s") * 2 + lax.axis_index("c")

        def chunk_body(t, carry):
            q = wid + t * NW

            @pl.when(q < NCHUNKS)
            def _():
                pltpu.sync_copy(idx_hbm.at[q], idx_v)
                pltpu.sync_copy(w_hbm.at[q], w_v)
                cps = [
                    pltpu.async_copy(
                        signal_hbm.at[idx_v.at[s]], rows_v.at[s], sem)
                    for s in range(3)
                ]
                for cp in cps:
                    cp.wait()
                for g in range(C // 16):
                    wv = [w_v[s, pl.ds(g * 16, 16)] for s in range(3)]
                    for j in range(16):
                        r = g * 16 + j
                        for dd in range(D // 16):
                            sl = pl.ds(dd * 16, 16)
                            acc_v[r, sl] = (
                                wv[0][j] * rows_v[0, r, sl]
                                + wv[1][j] * rows_v[1, r, sl]
                                + wv[2][j] * rows_v[2, r, sl]
                            )
                pltpu.sync_copy(acc_v, out_hbm.at[pl.ds(q * C, C)])

            return carry

        lax.fori_loop(0, TPW, chunk_body, 0)

    return body(signal, idx3, w3)


BN = 400
KD = K * D
RD = KT * D_OUT


def _tc_body(x_ref, w_ref, g_ref, s_ref, o_ref):
    hi = lax.Precision.HIGHEST
    conv = jnp.dot(x_ref[...], w_ref[...],
                   preferred_element_type=jnp.float32,
                   precision=lax.Precision.DEFAULT)
    norms_b = jnp.dot(conv * conv, g_ref[...],
                      preferred_element_type=jnp.float32, precision=hi)
    rmax = jnp.max(norms_b, axis=1, keepdims=True)
    col_iota = lax.broadcasted_iota(jnp.int32, (BN, RD), 1)
    win_col = jnp.min(jnp.where(norms_b >= rmax, col_iota, RD),
                      axis=1, keepdims=True)
    masked = jnp.where(col_iota // D_OUT == win_col // D_OUT, conv, 0.0)
    sel = jnp.dot(masked, s_ref[...],
                  preferred_element_type=jnp.float32, precision=hi)
    o_ref[...] = jnp.maximum(sel, 0.0)


def _tc_conv(pullback2d, w_mat, g_mat, s_mat):
    return pl.pallas_call(
        _tc_body,
        grid=(N // BN,),
        in_specs=[
            pl.BlockSpec((BN, KD), lambda i: (i, 0)),
            pl.BlockSpec((KD, RD), lambda i: (0, 0)),
            pl.BlockSpec((RD, RD), lambda i: (0, 0)),
            pl.BlockSpec((RD, D_OUT), lambda i: (0, 0)),
        ],
        out_specs=pl.BlockSpec((BN, D_OUT), lambda i: (i, 0)),
        out_shape=jax.ShapeDtypeStruct((N, D_OUT), jnp.float32),
    )(pullback2d, w_mat, g_mat, s_mat)


def kernel(signal, bary_verts, bary_weights, kernel):
    idx3 = (bary_verts.reshape(NCHUNKS, C, 3).astype(jnp.int32)
            .transpose(0, 2, 1))
    w3 = bary_weights.reshape(NCHUNKS, C, 3).astype(jnp.float32).transpose(0, 2, 1)

    kv = np.arange(K)
    rad = kv // KT
    ang = kv % KT
    rot = np.arange(KT)
    ang_rot = (ang[None, :] + rot[:, None]) % KT
    ker = kernel[np.broadcast_to(rad[None, :], (KT, K)), ang_rot]
    w_mat = ker.transpose(1, 2, 0, 3).reshape(KD, RD)

    cols = np.arange(RD)
    g_mat = jnp.asarray((cols[:, None] // D_OUT == cols[None, :] // D_OUT),
                        dtype=jnp.float32)
    s_mat = jnp.asarray((cols[:, None] % D_OUT == np.arange(D_OUT)[None, :]),
                        dtype=jnp.float32)

    pullback = _sc_pullback(signal, idx3, w3)
    return _tc_conv(pullback.reshape(N, KD), w_mat, g_mat, s_mat)
